# baseline (speedup 1.0000x reference)
import functools
import jax
import jax.numpy as jnp
from jax import lax
from jax.experimental import pallas as pl
from jax.experimental.pallas import tpu as pltpu
from jax.experimental.pallas import tpu_sc as plsc

B, M, D, E = 2, 2048, 768, 8
T = B * M
TB = 2048
NB = T // TB
BPB = max(NB // B, 1)
BLK = 256
NR = 2 * T + E * BLK
NBLK = NR // BLK
NBLKP = 48
NW = 32
IPW = 2 * T // 16
OPW = NR // NW
ZPW = NR // 16
GCH = 64
HIGH = lax.Precision.HIGHEST
DEF = lax.Precision.DEFAULT


def _cv2(v):
    mean = jnp.mean(v)
    var1 = jnp.sum((v - mean) ** 2) / (E - 1)
    return var1 / (mean * mean + 1e-10)


def _gate_body(x_ref, wg_ref, eg_ref, gg_ref, pg_ref, cnt_ref, bexp_ref,
               imp_ref, load_ref, run_scr):
    i = pl.program_id(0)
    xb = x_ref[...].astype(jnp.bfloat16)
    logits = jnp.dot(xb, wg_ref[...].astype(jnp.bfloat16),
                     preferred_element_type=jnp.float32)
    eidx = lax.broadcasted_iota(jnp.int32, (TB, E), 1)
    m1 = jnp.max(logits, axis=1, keepdims=True)
    a1 = jnp.min(jnp.where(logits == m1, eidx, E), axis=1, keepdims=True)
    masked = jnp.where(eidx == a1, -jnp.inf, logits)
    m2 = jnp.max(masked, axis=1, keepdims=True)
    a2 = jnp.min(jnp.where(masked == m2, eidx, E), axis=1, keepdims=True)
    ed = jnp.exp(m2 - m1)
    g1 = 1.0 / (1.0 + ed)
    g2 = ed / (1.0 + ed)
    gates = (jnp.where(eidx == a1, g1, 0.0)
             + jnp.where(eidx == a2, g2, 0.0))
    imp_ref[0, 0, :] = jnp.sum(gates, axis=0)
    load_ref[0, 0, :] = jnp.sum((gates > 0.0).astype(jnp.float32), axis=0)

    @pl.when(i == 0)
    def _():
        run_scr[...] = jnp.zeros_like(run_scr)

    n = ((eidx == a1) | (eidx == a2)).astype(jnp.int32)
    c = n
    sh = 1
    while sh < TB:
        c = c + jnp.concatenate(
            [jnp.zeros((sh, E), jnp.int32), c[:TB - sh, :]], axis=0)
        sh *= 2
    cex = c - n + run_scr[...]
    p1 = jnp.sum(jnp.where(eidx == a1, cex, 0), axis=1, keepdims=True)
    p2 = jnp.sum(jnp.where(eidx == a2, cex, 0), axis=1, keepdims=True)
    eg_ref[...] = jnp.concatenate([a1, a2], axis=1)
    gg_ref[...] = jnp.concatenate([g1, g2], axis=1)
    pg_ref[...] = jnp.concatenate([p1, p2], axis=1)
    new_run = run_scr[...] + jnp.sum(n, axis=0, keepdims=True)
    run_scr[...] = new_run

    @pl.when(i == NB - 1)
    def _():
        cnt_ref[...] = jnp.concatenate(
            [new_run, jnp.zeros((1, 16 - E), jnp.int32)], axis=1)
        lgb = BLK.bit_length() - 1
        padded = jnp.maximum(
            jnp.left_shift(jnp.right_shift(new_run + (BLK - 1), lgb), lgb),
            BLK)
        ends = jnp.concatenate(
            [jnp.sum(padded[:, :e2 + 1], axis=1, keepdims=True)
             for e2 in range(E)], axis=1)
        jpos = lax.broadcasted_iota(jnp.int32, (NBLKP, E), 0) * BLK
        cmp = (jpos >= ends).astype(jnp.int32)
        bexp_ref[...] = jnp.minimum(jnp.sum(cmp, axis=1, keepdims=True),
                                    E - 1)


def _sc_route(ecat, gcat, pcat, cnt16, x_hbm,
              src_o, wrow_o, xs_o,
              cnt_v, ebuf, gbuf, pbuf, posbuf, tvalbuf,
              sidxbuf, wbuf, sidxc, rows_v, zbi, zbf,
              src_sh, w_sh, sem):
    wid = lax.axis_index("s") * 2 + lax.axis_index("c")
    pltpu.sync_copy(cnt16, cnt_v)
    cv = cnt_v[...]
    lgb = BLK.bit_length() - 1
    padded = jnp.maximum(
        jnp.left_shift(jnp.right_shift(cv + (BLK - 1), lgb), lgb), BLK)
    lane = lax.broadcasted_iota(jnp.int32, (16,), 0)
    c = padded
    for shl in (1, 2, 4, 8):
        shifted = c.at[jnp.maximum(lane - shl, 0)].get(
            mode="promise_in_bounds")
        c = c + jnp.where(lane >= shl, shifted, 0)
    offs = c - padded

    sid = lax.axis_index("s")
    for k2 in range(ZPW // 16):
        zbi[pl.ds(k2 * 16, 16)] = jnp.zeros((16,), jnp.int32)
        zbf[pl.ds(k2 * 16, 16)] = jnp.zeros((16,), jnp.float32)
    pltpu.sync_copy(zbi, src_sh.at[pl.ds(sid * ZPW, ZPW)])
    pltpu.sync_copy(zbf, w_sh.at[pl.ds(sid * ZPW, ZPW)])
    plsc.subcore_barrier()

    ibase = sid * IPW
    pltpu.sync_copy(ecat.at[pl.ds(ibase, IPW)], ebuf)
    pltpu.sync_copy(pcat.at[pl.ds(ibase, IPW)], pbuf)
    for j in range(IPW // 128):
        pltpu.sync_copy(gcat.at[pl.ds(ibase + j * 128, 128)], gbuf.at[j])
    for cch in range(IPW // 16):
        ev = ebuf[pl.ds(cch * 16, 16)]
        pv = pbuf[pl.ds(cch * 16, 16)]
        ov = offs.at[ev].get(mode="promise_in_bounds")
        row = cch // 8
        col = (cch % 8) * 16
        posbuf[row, pl.ds(col, 16)] = pv + ov
        tbase = jnp.full((16,), ibase + cch * 16, jnp.int32)
        tvalbuf[row, pl.ds(col, 16)] = jnp.right_shift(tbase + lane, 1)
    for j in range(IPW // 128):
        pltpu.sync_copy(tvalbuf.at[j], src_sh.at[posbuf.at[j]])
        pltpu.sync_copy(gbuf.at[j], w_sh.at[posbuf.at[j]])
    plsc.subcore_barrier()

    base = wid * OPW
    pltpu.sync_copy(src_sh.at[pl.ds(base, OPW)], sidxbuf)
    pltpu.sync_copy(w_sh.at[pl.ds(base, OPW)], wbuf)
    pltpu.sync_copy(sidxbuf, src_o.at[pl.ds(base, OPW)])
    pltpu.sync_copy(wbuf, wrow_o.at[pl.ds(base, OPW)])
    for ch in range(OPW // GCH):
        pltpu.sync_copy(src_sh.at[pl.ds(base + ch * GCH, GCH)], sidxc)
        pltpu.async_copy(x_hbm.at[sidxc], rows_v, sem).wait()
        pltpu.sync_copy(rows_v, xs_o.at[pl.ds(base + ch * GCH, GCH)])


def _group_fc1_body(s_ref, xs_ref, w1_ref, b1_ref, src_ref, w_ref, gh_ref):
    j = pl.program_id(0)
    prev = s_ref[jnp.maximum(j - 1, 0)]
    first = jnp.logical_or(j == 0, s_ref[j] != prev)
    h = jnp.dot(xs_ref[...], w1_ref[0], preferred_element_type=jnp.float32,
                precision=DEF)
    h = jnp.maximum(h + b1_ref[0], 0.0)
    w = w_ref[...]
    bflag = src_ref[...] >= M
    wb0 = jnp.where(bflag, 0.0, w)
    wb1 = jnp.where(bflag, w, 0.0)
    v0 = jnp.sum(h * wb0, axis=0)
    v1 = jnp.sum(h * wb1, axis=0)
    val = jnp.concatenate([v0[None, :], v1[None, :]], axis=0)[None]

    @pl.when(first)
    def _():
        gh_ref[...] = val

    @pl.when(jnp.logical_not(first))
    def _():
        gh_ref[...] += val


def _final_body(gh_ref, w2_ref, b2_ref, imp_ref, load_ref, yt_ref, hw_ref,
                hb_ref, lng_ref, lnb_ref, scores_ref, aux_ref, pred_ref,
                acc_ref):
    e = pl.program_id(0)

    @pl.when(e == 0)
    def _():
        acc_ref[...] = jnp.zeros_like(acc_ref)

    acc_ref[0:B, :] += jnp.dot(gh_ref[0], w2_ref[0],
                               preferred_element_type=jnp.float32,
                               precision=DEF)

    @pl.when(e == E - 1)
    def _():
        imp_blk = imp_ref[:, 0, :]
        load_blk = load_ref[:, 0, :]
        imp_b = jnp.reshape(imp_blk, (B, BPB, E)).sum(axis=1)
        importance = jnp.sum(imp_blk, axis=0)
        load = jnp.sum(load_blk, axis=0)
        aux = (_cv2(importance) + _cv2(load)) * 0.01
        aux_ref[...] = jnp.reshape(aux, (1, 1))

        mm = acc_ref[0:B, :] + jnp.dot(imp_b, b2_ref[...],
                                       preferred_element_type=jnp.float32,
                                       precision=HIGH)
        mu = jnp.mean(mm, axis=1, keepdims=True)
        var = jnp.mean((mm - mu) ** 2, axis=1, keepdims=True)
        fin = (mm - mu) * lax.rsqrt(var + 1e-5) * lng_ref[...] + lnb_ref[...]
        out = jnp.dot(fin, hw_ref[...], preferred_element_type=jnp.float32,
                      precision=HIGH) + hb_ref[...]
        scores = jax.nn.sigmoid(out)
        scores_ref[...] = scores
        pred_ref[...] = jnp.reshape(
            jnp.mean((scores - yt_ref[...]) ** 2), (1, 1))


def _stage1(x, w_gate):
    return pl.pallas_call(
        _gate_body,
        grid=(NB,),
        in_specs=[
            pl.BlockSpec((TB, D), lambda i: (i, 0)),
            pl.BlockSpec((D, E), lambda i: (0, 0)),
        ],
        out_specs=[
            pl.BlockSpec((TB, 2), lambda i: (i, 0)),
            pl.BlockSpec((TB, 2), lambda i: (i, 0)),
            pl.BlockSpec((TB, 2), lambda i: (i, 0)),
            pl.BlockSpec((1, 16), lambda i: (0, 0)),
            pl.BlockSpec((NBLKP, 1), lambda i: (0, 0)),
            pl.BlockSpec((1, 1, E), lambda i: (i, 0, 0)),
            pl.BlockSpec((1, 1, E), lambda i: (i, 0, 0)),
        ],
        out_shape=[
            jax.ShapeDtypeStruct((T, 2), jnp.int32),
            jax.ShapeDtypeStruct((T, 2), jnp.float32),
            jax.ShapeDtypeStruct((T, 2), jnp.int32),
            jax.ShapeDtypeStruct((1, 16), jnp.int32),
            jax.ShapeDtypeStruct((NBLKP, 1), jnp.int32),
            jax.ShapeDtypeStruct((NB, 1, E), jnp.float32),
            jax.ShapeDtypeStruct((NB, 1, E), jnp.float32),
        ],
        scratch_shapes=[pltpu.VMEM((1, E), jnp.int32)],
    )(x, w_gate)


@functools.lru_cache(maxsize=1)
def _make_stage2():
    @functools.partial(
        pl.kernel,
        mesh=plsc.VectorSubcoreMesh(core_axis_name="c", subcore_axis_name="s"),
        out_type=[
            jax.ShapeDtypeStruct((NR,), jnp.int32),
            jax.ShapeDtypeStruct((NR,), jnp.float32),
            jax.ShapeDtypeStruct((NR, D), jnp.float32),
        ],
        scratch_types=[
            pltpu.VMEM((16,), jnp.int32),
            pltpu.VMEM((IPW,), jnp.int32),
            pltpu.VMEM((IPW // 128, 128), jnp.float32),
            pltpu.VMEM((IPW,), jnp.int32),
            pltpu.VMEM((IPW // 128, 128), jnp.int32),
            pltpu.VMEM((IPW // 128, 128), jnp.int32),
            pltpu.VMEM((OPW,), jnp.int32),
            pltpu.VMEM((OPW,), jnp.float32),
            pltpu.VMEM((GCH,), jnp.int32),
            pltpu.VMEM((GCH, D), jnp.float32),
            pltpu.VMEM((ZPW,), jnp.int32),
            pltpu.VMEM((ZPW,), jnp.float32),
            pltpu.VMEM_SHARED((NR,), jnp.int32),
            pltpu.VMEM_SHARED((NR,), jnp.float32),
            pltpu.SemaphoreType.DMA,
        ],
    )
    def _stage2_kernel(ecat, gcat, pcat, cnt16, x_hbm,
                       src_o, wrow_o, xs_o, *scratch):
        _sc_route(ecat, gcat, pcat, cnt16, x_hbm, src_o, wrow_o, xs_o,
                  *scratch)

    return _stage2_kernel


def _stage2(ecat, gcat, pcat, cnt16, x_hbm):
    return _make_stage2()(ecat, gcat, pcat, cnt16, x_hbm)


def _stage3(bexp, xs, fc1_w, fc1_b3, srcc, wcol):
    grid_spec = pltpu.PrefetchScalarGridSpec(
        num_scalar_prefetch=1,
        grid=(NBLK,),
        in_specs=[
            pl.BlockSpec((BLK, D), lambda j, s: (j, 0)),
            pl.BlockSpec((1, D, D), lambda j, s: (s[j], 0, 0)),
            pl.BlockSpec((1, 1, D), lambda j, s: (s[j], 0, 0)),
            pl.BlockSpec((BLK, 1), lambda j, s: (j, 0)),
            pl.BlockSpec((BLK, 1), lambda j, s: (j, 0)),
        ],
        out_specs=pl.BlockSpec((1, B, D), lambda j, s: (s[j], 0, 0)),
    )
    return pl.pallas_call(
        _group_fc1_body,
        grid_spec=grid_spec,
        out_shape=jax.ShapeDtypeStruct((E, B, D), jnp.float32),
    )(bexp, xs, fc1_w, fc1_b3, srcc, wcol)


def _stage4(gh, fc2_w, fc2_b, imp_blk, load_blk, true_y, head_w, head_b,
            ln_g, ln_b):
    return pl.pallas_call(
        _final_body,
        grid=(E,),
        in_specs=[
            pl.BlockSpec((1, B, D), lambda e: (e, 0, 0)),
            pl.BlockSpec((1, D, D), lambda e: (e, 0, 0)),
            pl.BlockSpec((E, D), lambda e: (0, 0)),
            pl.BlockSpec((NB, 1, E), lambda e: (0, 0, 0)),
            pl.BlockSpec((NB, 1, E), lambda e: (0, 0, 0)),
            pl.BlockSpec((B, 1), lambda e: (0, 0)),
            pl.BlockSpec((D, 1), lambda e: (0, 0)),
            pl.BlockSpec((1, 1), lambda e: (0, 0)),
            pl.BlockSpec((1, D), lambda e: (0, 0)),
            pl.BlockSpec((1, D), lambda e: (0, 0)),
        ],
        out_specs=[
            pl.BlockSpec((B, 1), lambda e: (0, 0)),
            pl.BlockSpec((1, 1), lambda e: (0, 0)),
            pl.BlockSpec((1, 1), lambda e: (0, 0)),
        ],
        out_shape=[
            jax.ShapeDtypeStruct((B, 1), jnp.float32),
            jax.ShapeDtypeStruct((1, 1), jnp.float32),
            jax.ShapeDtypeStruct((1, 1), jnp.float32),
        ],
        scratch_shapes=[pltpu.VMEM((8, D), jnp.float32)],
    )(gh, fc2_w, fc2_b, imp_blk, load_blk, true_y,
      head_w, head_b, ln_g, ln_b)


def kernel(mm_embed, task_index, true_y, w_gate, fc1_w, fc1_b, fc2_w, fc2_b,
           head_w, head_b, ln_g, ln_b):
    x = mm_embed.reshape(T, D)
    eg, gg, pg, cnt16, bexp, imp_blk, load_blk = _stage1(x, w_gate)
    src, wrow, xs = _stage2(
        eg.reshape(2 * T), gg.reshape(2 * T), pg.reshape(2 * T),
        cnt16.reshape(16), x)
    gh = _stage3(bexp.reshape(NBLKP)[:NBLK], xs, fc1_w,
                 fc1_b.reshape(E, 1, D),
                 src.reshape(NR, 1), wrow.reshape(NR, 1))
    scores, aux, pred = _stage4(
        gh, fc2_w, fc2_b, imp_blk, load_blk, true_y,
        head_w, head_b.reshape(1, 1), ln_g.reshape(1, D), ln_b.reshape(1, D))
    return (scores, aux.reshape(()), pred.reshape(()))


# speedup vs baseline: 3.1142x; 3.1142x over previous
import jax
import jax.numpy as jnp
from jax import lax
from jax.experimental import pallas as pl
from jax.experimental.pallas import tpu as pltpu

B, M, D, E = 2, 2048, 768, 8
T = B * M
TB = 2048
NB = T // TB
BPB = NB // B
HIGH = lax.Precision.HIGHEST


def _cv2(v):
    mean = jnp.mean(v)
    var1 = jnp.sum((v - mean) ** 2) / (E - 1)
    return var1 / (mean * mean + 1e-10)


def _moe_body(x_ref, wg_ref, w1_ref, b1_ref, w2_ref, b2_ref, yt_ref,
              hw_ref, hb_ref, lng_ref, lnb_ref,
              scores_ref, aux_ref, pred_ref,
              gates_scr, imp_scr, load_scr, fold_scr, mm_scr):
    e = pl.program_id(0)
    i = pl.program_id(1)
    b = i // BPB
    xf = x_ref[pl.ds(i * TB, TB), :]

    @pl.when(jnp.logical_and(e == 0, i == 0))
    def _():
        mm_scr[...] = jnp.zeros_like(mm_scr)

    @pl.when(e == 0)
    def _():
        xb = xf.astype(jnp.bfloat16)
        logits = jnp.dot(xb, wg_ref[...].astype(jnp.bfloat16),
                         preferred_element_type=jnp.float32)
        eidx = lax.broadcasted_iota(jnp.int32, (TB, E), 1)
        m1 = jnp.max(logits, axis=1, keepdims=True)
        a1 = jnp.min(jnp.where(logits == m1, eidx, E), axis=1, keepdims=True)
        masked = jnp.where(eidx == a1, -jnp.inf, logits)
        m2 = jnp.max(masked, axis=1, keepdims=True)
        a2 = jnp.min(jnp.where(masked == m2, eidx, E), axis=1, keepdims=True)
        ed = jnp.exp(m2 - m1)
        g1 = 1.0 / (1.0 + ed)
        g2 = ed / (1.0 + ed)
        gates = (jnp.where(eidx == a1, g1, 0.0)
                 + jnp.where(eidx == a2, g2, 0.0))
        gates_scr[pl.ds(i * TB, TB), :] = gates
        imp_scr[pl.ds(i, 1), :] = jnp.sum(gates, axis=0)[None, :]
        load_scr[pl.ds(i, 1), :] = jnp.sum((gates > 0.0).astype(jnp.float32),
                                           axis=0)[None, :]

    @pl.when(i == 0)
    def _():
        fold_scr[...] = jnp.zeros_like(fold_scr)

    h = jnp.dot(xf, w1_ref[0], preferred_element_type=jnp.float32,
                precision=lax.Precision.DEFAULT)
    h = jnp.maximum(h + b1_ref[0], 0.0)
    gall = gates_scr[pl.ds(i * TB, TB), :]
    sel = lax.broadcasted_iota(jnp.int32, (TB, E), 1) == e
    gcol = jnp.sum(jnp.where(sel, gall, 0.0), axis=1, keepdims=True)
    vec = jnp.sum(h * gcol, axis=0)
    fold_scr[pl.ds(b, 1), :] += vec[None, :]

    @pl.when(i == NB - 1)
    def _():
        mm_scr[0:B, :] += jnp.dot(fold_scr[0:B, :], w2_ref[0],
                                  preferred_element_type=jnp.float32,
                                  precision=lax.Precision.DEFAULT)

    @pl.when(jnp.logical_and(e == E - 1, i == NB - 1))
    def _():
        imp_blk = imp_scr[...]
        load_blk = load_scr[...]
        imp_b = jnp.reshape(imp_blk, (B, BPB, E)).sum(axis=1)
        importance = jnp.sum(imp_blk, axis=0)
        load = jnp.sum(load_blk, axis=0)
        aux = (_cv2(importance) + _cv2(load)) * 0.01
        aux_ref[...] = jnp.reshape(aux, (1, 1))

        mm = mm_scr[0:B, :] + jnp.dot(imp_b, b2_ref[...],
                                      preferred_element_type=jnp.float32,
                                      precision=HIGH)
        mu = jnp.mean(mm, axis=1, keepdims=True)
        var = jnp.mean((mm - mu) ** 2, axis=1, keepdims=True)
        fin = (mm - mu) * lax.rsqrt(var + 1e-5) * lng_ref[...] + lnb_ref[...]
        out = jnp.dot(fin, hw_ref[...], preferred_element_type=jnp.float32,
                      precision=HIGH) + hb_ref[...]
        scores = jax.nn.sigmoid(out)
        scores_ref[...] = scores
        pred_ref[...] = jnp.reshape(
            jnp.mean((scores - yt_ref[...]) ** 2), (1, 1))


def kernel(mm_embed, task_index, true_y, w_gate, fc1_w, fc1_b, fc2_w, fc2_b,
           head_w, head_b, ln_g, ln_b):
    x = mm_embed.reshape(T, D)

    scores, aux, pred = pl.pallas_call(
        _moe_body,
        grid=(E, NB),
        in_specs=[
            pl.BlockSpec((T, D), lambda e, i: (0, 0)),
            pl.BlockSpec((D, E), lambda e, i: (0, 0)),
            pl.BlockSpec((1, D, D), lambda e, i: (e, 0, 0)),
            pl.BlockSpec((1, 1, D), lambda e, i: (e, 0, 0)),
            pl.BlockSpec((1, D, D), lambda e, i: (e, 0, 0)),
            pl.BlockSpec((E, D), lambda e, i: (0, 0)),
            pl.BlockSpec((B, 1), lambda e, i: (0, 0)),
            pl.BlockSpec((D, 1), lambda e, i: (0, 0)),
            pl.BlockSpec((1, 1), lambda e, i: (0, 0)),
            pl.BlockSpec((1, D), lambda e, i: (0, 0)),
            pl.BlockSpec((1, D), lambda e, i: (0, 0)),
        ],
        out_specs=[
            pl.BlockSpec((B, 1), lambda e, i: (0, 0)),
            pl.BlockSpec((1, 1), lambda e, i: (0, 0)),
            pl.BlockSpec((1, 1), lambda e, i: (0, 0)),
        ],
        out_shape=[
            jax.ShapeDtypeStruct((B, 1), jnp.float32),
            jax.ShapeDtypeStruct((1, 1), jnp.float32),
            jax.ShapeDtypeStruct((1, 1), jnp.float32),
        ],
        scratch_shapes=[
            pltpu.VMEM((T, E), jnp.float32),
            pltpu.VMEM((NB, E), jnp.float32),
            pltpu.VMEM((NB, E), jnp.float32),
            pltpu.VMEM((8, D), jnp.float32),
            pltpu.VMEM((8, D), jnp.float32),
        ],
    )(x, w_gate, fc1_w, fc1_b.reshape(E, 1, D), fc2_w, fc2_b, true_y,
      head_w, head_b.reshape(1, 1), ln_g.reshape(1, D), ln_b.reshape(1, D))

    return (scores, aux.reshape(()), pred.reshape(()))
